# baseline (speedup 1.0000x reference)
import functools

import jax
import jax.numpy as jnp
from jax import lax
from jax.experimental import pallas as pl
from jax.experimental.pallas import tpu as pltpu
from jax.experimental.pallas import tpu_sc as plsc

S = 384
CZ = 256
VBINS = 2 * (S - 1) + 1
VPAD = 768
APAD = 776


def _emb8_body(w_ref, b_ref, out_ref):
    t = w_ref[...].T + b_ref[...]
    for k in range(8):
        out_ref[k, pl.ds(k, VPAD), :] = t


def _build_emb8(w_pad, b2):
    return pl.pallas_call(
        _emb8_body,
        out_shape=jax.ShapeDtypeStruct((8, APAD, CZ), jnp.float32),
    )(w_pad, b2)


def _make_sc_writer():
    info = plsc.get_sparse_core_info()
    nc, ns = info.num_cores, info.num_subcores
    nw = nc * ns
    rows_per_w = S // nw
    mesh = plsc.VectorSubcoreMesh(core_axis_name="c", subcore_axis_name="s")

    @functools.partial(
        pl.kernel,
        mesh=mesh,
        out_type=jax.ShapeDtypeStruct((S, S, CZ), jnp.float32),
        scratch_types=[
            pltpu.VMEM_SHARED((8, APAD, CZ), jnp.float32),
            pltpu.SemaphoreType.DMA,
        ],
    )
    def sc_writer(emb8_hbm, out_hbm, spmem, sem):
        cid = lax.axis_index("c")
        sid = lax.axis_index("s")
        half = sid % 2
        kcopy = sid // 2
        r0 = pl.multiple_of(half * 384, 8)
        pltpu.sync_copy(
            emb8_hbm.at[kcopy, pl.ds(r0, 392), :],
            spmem.at[kcopy, pl.ds(r0, 392), :],
        )
        plsc.subcore_barrier()
        wid = sid * nc + cid
        copies = []
        for r in range(rows_per_w):
            i = wid * rows_per_w + r
            v = (S - 1) - i
            k = (8 - lax.rem(v, 8)) % 8
            off = pl.multiple_of(v + k, 8)
            copies.append(
                pltpu.async_copy(
                    spmem.at[k, pl.ds(off, S), :],
                    out_hbm.at[i],
                    sem,
                )
            )
        for c in copies:
            c.wait()

    return sc_writer


_SC_WRITER = None


def _get_sc_writer():
    global _SC_WRITER
    if _SC_WRITER is None:
        _SC_WRITER = _make_sc_writer()
    return _SC_WRITER


def kernel(seq_len, ResInd, Wp_w, Wp_b):
    sc_writer = _get_sc_writer()
    w_pad = jnp.pad(Wp_w, ((0, 0), (0, VPAD - VBINS)))
    emb8 = _build_emb8(w_pad, Wp_b.reshape(1, CZ))
    return sc_writer(emb8)
